# Initial kernel scaffold; baseline (speedup 1.0000x reference)
import jax
import jax.numpy as jnp
from jax.experimental import pallas as pl


def kernel(x, edge_index_0, edge_index_1, W_l0, b_l0, W_r0, W_l1, b_l1, W_r1):
    raise NotImplementedError("write your pallas kernel here")


# speedup vs baseline: 4.5831x; 4.5831x over previous
import functools

import jax
import jax.numpy as jnp
from jax import lax
from jax.experimental import pallas as pl
from jax.experimental.pallas import tpu as pltpu
from jax.experimental.pallas import tpu_sc as plsc

N0 = 262144
N1 = 16384
N2 = 1024
E0 = 262144
E1 = 16384
D = 128
H = 64
NC = 2
NS = 16
G = 128

_mesh = lambda: plsc.VectorSubcoreMesh(
    core_axis_name="c", subcore_axis_name="s", num_cores=NC, num_subcores=NS)
_SC_PARAMS = pltpu.CompilerParams(use_tc_tiling_on_sc=False)


_L0_ROWS = E0 // G // NS
_L0_CH = 16


def _l0_agg_body(x2, src, dst, ones, z64, z16, s2_out, cnt_out,
                 srcbuf, dstbuf, rowbuf, onesbuf, acc, cntacc, sem):
    c = lax.axis_index("c")
    s = lax.axis_index("s")
    base = s * _L0_ROWS
    pltpu.sync_copy(ones, onesbuf)
    stripe = N1 // NS
    pltpu.sync_copy(z64, acc.at[pl.ds(s * stripe, stripe)])
    pltpu.sync_copy(z16, cntacc.at[pl.ds(s * stripe, stripe)])
    plsc.subcore_barrier()

    def chunk(k, carry):
        pltpu.sync_copy(src.at[pl.ds(base + k * _L0_CH, _L0_CH)], srcbuf)
        pltpu.sync_copy(dst.at[pl.ds(base + k * _L0_CH, _L0_CH)], dstbuf)

        def idx_body(r, cc):
            for j in range(G // 16):
                sl = pl.ds(j * 16, 16)
                srcbuf[r, sl] = srcbuf[r, sl] * 2 + c
            return cc

        lax.fori_loop(0, _L0_CH, idx_body, 0)

        def body(r, cc):
            pltpu.async_copy(x2.at[srcbuf.at[r]], rowbuf, sem).wait()
            pltpu.sync_copy(rowbuf, acc.at[dstbuf.at[r]], add=True)

            @pl.when(c == 0)
            def _():
                pltpu.sync_copy(onesbuf, cntacc.at[dstbuf.at[r]], add=True)

            return cc

        lax.fori_loop(0, _L0_CH, body, 0)
        return carry

    lax.fori_loop(0, _L0_ROWS // _L0_CH, chunk, 0)
    plsc.subcore_barrier()
    pltpu.sync_copy(acc.at[pl.ds(s * stripe, stripe)],
                    s2_out.at[c, pl.ds(s * stripe, stripe)])

    @pl.when(c == 0)
    def _():
        pltpu.sync_copy(cntacc.at[pl.ds(s * stripe, stripe)],
                        cnt_out.at[pl.ds(s * stripe, stripe)])


def _l0_agg(x2, src, dst, ones, z64, z16):
    f = functools.partial(
        pl.kernel,
        out_type=(jax.ShapeDtypeStruct((NC, N1, H), jnp.float32),
                  jax.ShapeDtypeStruct((N1, 16), jnp.float32)),
        mesh=_mesh(),
        scratch_types=[
            pltpu.VMEM((_L0_CH, G), jnp.int32),
            pltpu.VMEM((_L0_CH, G), jnp.int32),
            pltpu.VMEM((G, H), jnp.float32),
            pltpu.VMEM((G, 16), jnp.float32),
            pltpu.VMEM_SHARED((N1, H), jnp.float32),
            pltpu.VMEM_SHARED((N1, 16), jnp.float32),
            pltpu.SemaphoreType.DMA,
        ],
        compiler_params=_SC_PARAMS,
    )(_l0_agg_body)
    return f(x2, src, dst, ones, z64, z16)


_B0 = 2048


def _l0_dense_body(s2_ref, cnt_ref, xd_ref, wl_ref, wr_ref, b_ref, out_ref):
    inv = 1.0 / jnp.maximum(cnt_ref[:, 0:1], 1.0)
    wl = wl_ref[...]
    acc = jnp.dot(s2_ref[0] * inv, wl[:H], preferred_element_type=jnp.float32)
    acc += jnp.dot(s2_ref[1] * inv, wl[H:], preferred_element_type=jnp.float32)
    acc += jnp.dot(xd_ref[...], wr_ref[...], preferred_element_type=jnp.float32)
    out_ref[...] = jnp.maximum(acc + b_ref[...], 0.0)


def _l0_dense(s2, cnt, x_dst, W_l, W_r, b):
    grid = N1 // _B0
    return pl.pallas_call(
        _l0_dense_body,
        grid=(grid,),
        in_specs=[
            pl.BlockSpec((NC, _B0, H), lambda i: (0, i, 0)),
            pl.BlockSpec((_B0, 16), lambda i: (i, 0)),
            pl.BlockSpec((_B0, D), lambda i: (i, 0)),
            pl.BlockSpec((D, D), lambda i: (0, 0)),
            pl.BlockSpec((D, D), lambda i: (0, 0)),
            pl.BlockSpec((1, D), lambda i: (0, 0)),
        ],
        out_specs=pl.BlockSpec((_B0, D), lambda i: (i, 0)),
        out_shape=jax.ShapeDtypeStruct((N1, D), jnp.float32),
    )(s2, cnt, x_dst, W_l, W_r, b)


_L1_ROWS = E1 // G // (NC * NS)


def _l1_agg_body(h, src, dst, ones, z128, z16c, s1_out, cnt_out,
                 srcbuf, dstbuf, rowbuf, onesbuf, acc, cntacc, sem):
    c = lax.axis_index("c")
    s = lax.axis_index("s")
    w = s * NC + c
    pltpu.sync_copy(src.at[pl.ds(w * _L1_ROWS, _L1_ROWS)], srcbuf)
    pltpu.sync_copy(dst.at[pl.ds(w * _L1_ROWS, _L1_ROWS)], dstbuf)
    pltpu.sync_copy(ones, onesbuf)
    stripe = N2 // NS
    pltpu.sync_copy(z128, acc.at[pl.ds(s * stripe, stripe)])
    pltpu.sync_copy(z16c, cntacc.at[pl.ds(s * stripe, stripe)])
    plsc.subcore_barrier()
    for r in range(_L1_ROWS):
        pltpu.async_copy(h.at[srcbuf.at[r]], rowbuf, sem).wait()
        pltpu.sync_copy(rowbuf, acc.at[dstbuf.at[r]], add=True)
        pltpu.sync_copy(onesbuf, cntacc.at[dstbuf.at[r]], add=True)
    plsc.subcore_barrier()
    pltpu.sync_copy(acc.at[pl.ds(s * stripe, stripe)],
                    s1_out.at[c, pl.ds(s * stripe, stripe)])
    pltpu.sync_copy(cntacc.at[pl.ds(s * stripe, stripe)],
                    cnt_out.at[c, pl.ds(s * stripe, stripe)])


def _l1_agg(h, src, dst, ones, z128, z16c):
    f = functools.partial(
        pl.kernel,
        out_type=(jax.ShapeDtypeStruct((NC, N2, D), jnp.float32),
                  jax.ShapeDtypeStruct((NC, N2, 16), jnp.float32)),
        mesh=_mesh(),
        scratch_types=[
            pltpu.VMEM((_L1_ROWS, G), jnp.int32),
            pltpu.VMEM((_L1_ROWS, G), jnp.int32),
            pltpu.VMEM((G, D), jnp.float32),
            pltpu.VMEM((G, 16), jnp.float32),
            pltpu.VMEM_SHARED((N2, D), jnp.float32),
            pltpu.VMEM_SHARED((N2, 16), jnp.float32),
            pltpu.SemaphoreType.DMA,
        ],
        compiler_params=_SC_PARAMS,
    )(_l1_agg_body)
    return f(h, src, dst, ones, z128, z16c)


def _l1_dense_body(s1_ref, cnt_ref, hd_ref, wl_ref, wr_ref, b_ref, out_ref):
    summed = s1_ref[0] + s1_ref[1]
    cnt = cnt_ref[0][:, 0:1] + cnt_ref[1][:, 0:1]
    mean = summed / jnp.maximum(cnt, 1.0)
    o = jnp.dot(mean, wl_ref[...], preferred_element_type=jnp.float32)
    o += jnp.dot(hd_ref[...], wr_ref[...], preferred_element_type=jnp.float32)
    o += b_ref[...]
    m = jnp.max(o, axis=-1, keepdims=True)
    e = jnp.exp(o - m)
    lse = jnp.log(jnp.sum(e, axis=-1, keepdims=True))
    out_ref[...] = o - m - lse


def _l1_dense(s1, cnt1, h_dst, W_l, W_r, b):
    return pl.pallas_call(
        _l1_dense_body,
        out_shape=jax.ShapeDtypeStruct((N2, D), jnp.float32),
    )(s1, cnt1, h_dst, W_l, W_r, b)


def kernel(x, edge_index_0, edge_index_1, W_l0, b_l0, W_r0, W_l1, b_l1, W_r1):
    x2 = x.reshape(2 * N0, H)
    src0 = edge_index_0[0].astype(jnp.int32).reshape(E0 // G, G)
    dst0 = edge_index_0[1].astype(jnp.int32).reshape(E0 // G, G)
    src1 = edge_index_1[0].astype(jnp.int32).reshape(E1 // G, G)
    dst1 = edge_index_1[1].astype(jnp.int32).reshape(E1 // G, G)
    ones = jnp.ones((G, 16), jnp.float32)
    z64 = jnp.zeros((N1 // NS, H), jnp.float32)
    z16 = jnp.zeros((N1 // NS, 16), jnp.float32)
    z128 = jnp.zeros((N2 // NS, D), jnp.float32)
    z16c = jnp.zeros((N2 // NS, 16), jnp.float32)

    s2, cnt0 = _l0_agg(x2, src0, dst0, ones, z64, z16)
    h = _l0_dense(s2, cnt0, x[:N1], W_l0, W_r0, b_l0.reshape(1, D))
    s1, cnt1 = _l1_agg(h, src1, dst1, ones, z128, z16c)
    return _l1_dense(s1, cnt1, h[:N2], W_l1, W_r1, b_l1.reshape(1, D))


# speedup vs baseline: 6.1309x; 1.3377x over previous
import functools

import jax
import jax.numpy as jnp
from jax import lax
from jax.experimental import pallas as pl
from jax.experimental.pallas import tpu as pltpu
from jax.experimental.pallas import tpu_sc as plsc

N0 = 262144
N1 = 16384
N2 = 1024
E0 = 262144
E1 = 16384
D = 128
H = 64
NC = 2
NS = 16
G = 128

_mesh = lambda: plsc.VectorSubcoreMesh(
    core_axis_name="c", subcore_axis_name="s", num_cores=NC, num_subcores=NS)
_SC_PARAMS = pltpu.CompilerParams(use_tc_tiling_on_sc=False)


_L0_ROWS = E0 // G // NS
_L0_CH = 16
_NBUF = 2


def _l0_agg_body(x2, src, dst, ones, z64, z16, s2_out, cnt_out,
                 srcbuf, dstbuf, rowbuf, onesbuf, acc, cntacc, sem0, sem1):
    c = lax.axis_index("c")
    s = lax.axis_index("s")
    base = s * _L0_ROWS
    sems = (sem0, sem1)
    pltpu.sync_copy(ones, onesbuf)
    stripe = N1 // NS
    pltpu.sync_copy(z64, acc.at[pl.ds(s * stripe, stripe)])
    pltpu.sync_copy(z16, cntacc.at[pl.ds(s * stripe, stripe)])
    plsc.subcore_barrier()

    def chunk(k, carry):
        pltpu.sync_copy(src.at[pl.ds(base + k * _L0_CH, _L0_CH)], srcbuf)
        pltpu.sync_copy(dst.at[pl.ds(base + k * _L0_CH, _L0_CH)], dstbuf)

        def idx_body(r, cc):
            for j in range(G // 16):
                sl = pl.ds(j * 16, 16)
                srcbuf[r, sl] = srcbuf[r, sl] * 2 + c
            return cc

        lax.fori_loop(0, _L0_CH, idx_body, 0)

        for b in range(_NBUF):
            pltpu.async_copy(x2.at[srcbuf.at[b]], rowbuf.at[b], sems[b])

        def body(g, cc):
            for b in range(_NBUF):
                r = g * _NBUF + b
                pltpu.make_async_copy(
                    x2.at[pl.ds(0, G)], rowbuf.at[b], sems[b]).wait()
                pltpu.sync_copy(rowbuf.at[b], acc.at[dstbuf.at[r]], add=True)

                @pl.when(c == b)
                def _():
                    pltpu.sync_copy(onesbuf, cntacc.at[dstbuf.at[r]],
                                    add=True)

                @pl.when(r + _NBUF < _L0_CH)
                def _():
                    pltpu.async_copy(
                        x2.at[srcbuf.at[r + _NBUF]], rowbuf.at[b], sems[b])

            return cc

        lax.fori_loop(0, _L0_CH // _NBUF, body, 0)
        return carry

    lax.fori_loop(0, _L0_ROWS // _L0_CH, chunk, 0)
    plsc.subcore_barrier()
    pltpu.sync_copy(acc.at[pl.ds(s * stripe, stripe)],
                    s2_out.at[c, pl.ds(s * stripe, stripe)])
    pltpu.sync_copy(cntacc.at[pl.ds(s * stripe, stripe)],
                    cnt_out.at[c, pl.ds(s * stripe, stripe)])


def _l0_agg(x2, src, dst, ones, z64, z16):
    f = functools.partial(
        pl.kernel,
        out_type=(jax.ShapeDtypeStruct((NC, N1, H), jnp.float32),
                  jax.ShapeDtypeStruct((NC, N1, 16), jnp.float32)),
        mesh=_mesh(),
        scratch_types=[
            pltpu.VMEM((_L0_CH, G), jnp.int32),
            pltpu.VMEM((_L0_CH, G), jnp.int32),
            pltpu.VMEM((_NBUF, G, H), jnp.float32),
            pltpu.VMEM((G, 16), jnp.float32),
            pltpu.VMEM_SHARED((N1, H), jnp.float32),
            pltpu.VMEM_SHARED((N1, 16), jnp.float32),
            pltpu.SemaphoreType.DMA,
            pltpu.SemaphoreType.DMA,
        ],
        compiler_params=_SC_PARAMS,
    )(_l0_agg_body)
    return f(x2, src, dst, ones, z64, z16)


_B0 = 2048


def _l0_dense_body(s2_ref, cnt_ref, xd_ref, wl_ref, wr_ref, b_ref, out_ref):
    cnt = cnt_ref[0][:, 0:1] + cnt_ref[1][:, 0:1]
    inv = 1.0 / jnp.maximum(cnt, 1.0)
    wl = wl_ref[...]
    acc = jnp.dot(s2_ref[0] * inv, wl[:H], preferred_element_type=jnp.float32)
    acc += jnp.dot(s2_ref[1] * inv, wl[H:], preferred_element_type=jnp.float32)
    acc += jnp.dot(xd_ref[...], wr_ref[...], preferred_element_type=jnp.float32)
    out_ref[...] = jnp.maximum(acc + b_ref[...], 0.0)


def _l0_dense(s2, cnt, x_dst, W_l, W_r, b):
    grid = N1 // _B0
    return pl.pallas_call(
        _l0_dense_body,
        grid=(grid,),
        in_specs=[
            pl.BlockSpec((NC, _B0, H), lambda i: (0, i, 0)),
            pl.BlockSpec((NC, _B0, 16), lambda i: (0, i, 0)),
            pl.BlockSpec((_B0, D), lambda i: (i, 0)),
            pl.BlockSpec((D, D), lambda i: (0, 0)),
            pl.BlockSpec((D, D), lambda i: (0, 0)),
            pl.BlockSpec((1, D), lambda i: (0, 0)),
        ],
        out_specs=pl.BlockSpec((_B0, D), lambda i: (i, 0)),
        out_shape=jax.ShapeDtypeStruct((N1, D), jnp.float32),
    )(s2, cnt, x_dst, W_l, W_r, b)


_L1_ROWS = E1 // G // (NC * NS)


def _l1_agg_body(h, src, dst, ones, z128, z16c, s1_out, cnt_out,
                 srcbuf, dstbuf, rowbuf, onesbuf, acc, cntacc, sem):
    c = lax.axis_index("c")
    s = lax.axis_index("s")
    w = s * NC + c
    pltpu.sync_copy(src.at[pl.ds(w * _L1_ROWS, _L1_ROWS)], srcbuf)
    pltpu.sync_copy(dst.at[pl.ds(w * _L1_ROWS, _L1_ROWS)], dstbuf)
    pltpu.sync_copy(ones, onesbuf)
    stripe = N2 // NS
    pltpu.sync_copy(z128, acc.at[pl.ds(s * stripe, stripe)])
    pltpu.sync_copy(z16c, cntacc.at[pl.ds(s * stripe, stripe)])
    plsc.subcore_barrier()
    for r in range(_L1_ROWS):
        pltpu.async_copy(h.at[srcbuf.at[r]], rowbuf, sem).wait()
        pltpu.sync_copy(rowbuf, acc.at[dstbuf.at[r]], add=True)
        pltpu.sync_copy(onesbuf, cntacc.at[dstbuf.at[r]], add=True)
    plsc.subcore_barrier()
    pltpu.sync_copy(acc.at[pl.ds(s * stripe, stripe)],
                    s1_out.at[c, pl.ds(s * stripe, stripe)])
    pltpu.sync_copy(cntacc.at[pl.ds(s * stripe, stripe)],
                    cnt_out.at[c, pl.ds(s * stripe, stripe)])


def _l1_agg(h, src, dst, ones, z128, z16c):
    f = functools.partial(
        pl.kernel,
        out_type=(jax.ShapeDtypeStruct((NC, N2, D), jnp.float32),
                  jax.ShapeDtypeStruct((NC, N2, 16), jnp.float32)),
        mesh=_mesh(),
        scratch_types=[
            pltpu.VMEM((_L1_ROWS, G), jnp.int32),
            pltpu.VMEM((_L1_ROWS, G), jnp.int32),
            pltpu.VMEM((G, D), jnp.float32),
            pltpu.VMEM((G, 16), jnp.float32),
            pltpu.VMEM_SHARED((N2, D), jnp.float32),
            pltpu.VMEM_SHARED((N2, 16), jnp.float32),
            pltpu.SemaphoreType.DMA,
        ],
        compiler_params=_SC_PARAMS,
    )(_l1_agg_body)
    return f(h, src, dst, ones, z128, z16c)


def _l1_dense_body(s1_ref, cnt_ref, hd_ref, wl_ref, wr_ref, b_ref, out_ref):
    summed = s1_ref[0] + s1_ref[1]
    cnt = cnt_ref[0][:, 0:1] + cnt_ref[1][:, 0:1]
    mean = summed / jnp.maximum(cnt, 1.0)
    o = jnp.dot(mean, wl_ref[...], preferred_element_type=jnp.float32)
    o += jnp.dot(hd_ref[...], wr_ref[...], preferred_element_type=jnp.float32)
    o += b_ref[...]
    m = jnp.max(o, axis=-1, keepdims=True)
    e = jnp.exp(o - m)
    lse = jnp.log(jnp.sum(e, axis=-1, keepdims=True))
    out_ref[...] = o - m - lse


def _l1_dense(s1, cnt1, h_dst, W_l, W_r, b):
    return pl.pallas_call(
        _l1_dense_body,
        out_shape=jax.ShapeDtypeStruct((N2, D), jnp.float32),
    )(s1, cnt1, h_dst, W_l, W_r, b)


def kernel(x, edge_index_0, edge_index_1, W_l0, b_l0, W_r0, W_l1, b_l1, W_r1):
    x2 = x.reshape(2 * N0, H)
    src0 = edge_index_0[0].astype(jnp.int32).reshape(E0 // G, G)
    dst0 = edge_index_0[1].astype(jnp.int32).reshape(E0 // G, G)
    src1 = edge_index_1[0].astype(jnp.int32).reshape(E1 // G, G)
    dst1 = edge_index_1[1].astype(jnp.int32).reshape(E1 // G, G)
    ones = jnp.ones((G, 16), jnp.float32)
    z64 = jnp.zeros((N1 // NS, H), jnp.float32)
    z16 = jnp.zeros((N1 // NS, 16), jnp.float32)
    z128 = jnp.zeros((N2 // NS, D), jnp.float32)
    z16c = jnp.zeros((N2 // NS, 16), jnp.float32)

    s2, cnt0 = _l0_agg(x2, src0, dst0, ones, z64, z16)
    h = _l0_dense(s2, cnt0, x[:N1], W_l0, W_r0, b_l0.reshape(1, D))
    s1, cnt1 = _l1_agg(h, src1, dst1, ones, z128, z16c)
    return _l1_dense(s1, cnt1, h[:N2], W_l1, W_r1, b_l1.reshape(1, D))


# speedup vs baseline: 7.1424x; 1.1650x over previous
import functools

import jax
import jax.numpy as jnp
from jax import lax
from jax.experimental import pallas as pl
from jax.experimental.pallas import tpu as pltpu
from jax.experimental.pallas import tpu_sc as plsc

N0 = 262144
N1 = 16384
N2 = 1024
E0 = 262144
E1 = 16384
D = 128
H = 64
NC = 2
NS = 16
G = 128

_mesh = lambda: plsc.VectorSubcoreMesh(
    core_axis_name="c", subcore_axis_name="s", num_cores=NC, num_subcores=NS)
_SC_PARAMS = pltpu.CompilerParams(use_tc_tiling_on_sc=False)


_L0_ROWS = E0 // G // NS
_L0_CH = 32
_NBUF = 4


def _l0_agg_body(x2, src, dst, ones, z64, z16, s2_out, cnt_out,
                 srcbuf, dstbuf, rowbuf, onesbuf, acc, cntacc,
                 sem0, sem1, sem2, sem3):
    c = lax.axis_index("c")
    s = lax.axis_index("s")
    base = s * _L0_ROWS
    sems = (sem0, sem1, sem2, sem3)
    pltpu.sync_copy(ones, onesbuf)
    stripe = N1 // NS
    pltpu.sync_copy(z64, acc.at[pl.ds(s * stripe, stripe)])
    pltpu.sync_copy(z16, cntacc.at[pl.ds(s * stripe, stripe)])
    plsc.subcore_barrier()

    def chunk(k, carry):
        pltpu.sync_copy(src.at[pl.ds(base + k * _L0_CH, _L0_CH)], srcbuf)
        pltpu.sync_copy(dst.at[pl.ds(base + k * _L0_CH, _L0_CH)], dstbuf)

        def idx_body(r, cc):
            for j in range(G // 16):
                sl = pl.ds(j * 16, 16)
                srcbuf[r, sl] = srcbuf[r, sl] * 2 + c
            return cc

        lax.fori_loop(0, _L0_CH, idx_body, 0)

        for b in range(_NBUF):
            pltpu.async_copy(x2.at[srcbuf.at[b]], rowbuf.at[b], sems[b])

        def body(g, cc):
            for b in range(_NBUF):
                r = g * _NBUF + b
                pltpu.make_async_copy(
                    x2.at[pl.ds(0, G)], rowbuf.at[b], sems[b]).wait()
                pltpu.sync_copy(rowbuf.at[b], acc.at[dstbuf.at[r]], add=True)

                @pl.when(c == b % 2)
                def _():
                    pltpu.sync_copy(onesbuf, cntacc.at[dstbuf.at[r]],
                                    add=True)

                @pl.when(r + _NBUF < _L0_CH)
                def _():
                    pltpu.async_copy(
                        x2.at[srcbuf.at[r + _NBUF]], rowbuf.at[b], sems[b])

            return cc

        lax.fori_loop(0, _L0_CH // _NBUF, body, 0)
        return carry

    lax.fori_loop(0, _L0_ROWS // _L0_CH, chunk, 0)
    plsc.subcore_barrier()
    pltpu.sync_copy(acc.at[pl.ds(s * stripe, stripe)],
                    s2_out.at[c, pl.ds(s * stripe, stripe)])
    pltpu.sync_copy(cntacc.at[pl.ds(s * stripe, stripe)],
                    cnt_out.at[c, pl.ds(s * stripe, stripe)])


def _l0_agg(x2, src, dst, ones, z64, z16):
    f = functools.partial(
        pl.kernel,
        out_type=(jax.ShapeDtypeStruct((NC, N1, H), jnp.float32),
                  jax.ShapeDtypeStruct((NC, N1, 16), jnp.float32)),
        mesh=_mesh(),
        scratch_types=[
            pltpu.VMEM((_L0_CH, G), jnp.int32),
            pltpu.VMEM((_L0_CH, G), jnp.int32),
            pltpu.VMEM((_NBUF, G, H), jnp.float32),
            pltpu.VMEM((G, 16), jnp.float32),
            pltpu.VMEM_SHARED((N1, H), jnp.float32),
            pltpu.VMEM_SHARED((N1, 16), jnp.float32),
            pltpu.SemaphoreType.DMA,
            pltpu.SemaphoreType.DMA,
            pltpu.SemaphoreType.DMA,
            pltpu.SemaphoreType.DMA,
        ],
        compiler_params=_SC_PARAMS,
    )(_l0_agg_body)
    return f(x2, src, dst, ones, z64, z16)


_B0 = 2048


def _l0_dense_body(s2_ref, cnt_ref, xd_ref, wl_ref, wr_ref, b_ref, out_ref):
    cnt = cnt_ref[0][:, 0:1] + cnt_ref[1][:, 0:1]
    inv = 1.0 / jnp.maximum(cnt, 1.0)
    wl = wl_ref[...]
    acc = jnp.dot(s2_ref[0] * inv, wl[:H], preferred_element_type=jnp.float32)
    acc += jnp.dot(s2_ref[1] * inv, wl[H:], preferred_element_type=jnp.float32)
    acc += jnp.dot(xd_ref[...], wr_ref[...], preferred_element_type=jnp.float32)
    out_ref[...] = jnp.maximum(acc + b_ref[...], 0.0)


def _l0_dense(s2, cnt, x_dst, W_l, W_r, b):
    grid = N1 // _B0
    return pl.pallas_call(
        _l0_dense_body,
        grid=(grid,),
        in_specs=[
            pl.BlockSpec((NC, _B0, H), lambda i: (0, i, 0)),
            pl.BlockSpec((NC, _B0, 16), lambda i: (0, i, 0)),
            pl.BlockSpec((_B0, D), lambda i: (i, 0)),
            pl.BlockSpec((D, D), lambda i: (0, 0)),
            pl.BlockSpec((D, D), lambda i: (0, 0)),
            pl.BlockSpec((1, D), lambda i: (0, 0)),
        ],
        out_specs=pl.BlockSpec((_B0, D), lambda i: (i, 0)),
        out_shape=jax.ShapeDtypeStruct((N1, D), jnp.float32),
    )(s2, cnt, x_dst, W_l, W_r, b)


_L1_ROWS = E1 // G // (NC * NS)


def _l1_agg_body(h, src, dst, ones, z128, z16c, s1_out, cnt_out,
                 srcbuf, dstbuf, rowbuf, onesbuf, acc, cntacc, sem):
    c = lax.axis_index("c")
    s = lax.axis_index("s")
    w = s * NC + c
    pltpu.sync_copy(src.at[pl.ds(w * _L1_ROWS, _L1_ROWS)], srcbuf)
    pltpu.sync_copy(dst.at[pl.ds(w * _L1_ROWS, _L1_ROWS)], dstbuf)
    pltpu.sync_copy(ones, onesbuf)
    stripe = N2 // NS
    pltpu.sync_copy(z128, acc.at[pl.ds(s * stripe, stripe)])
    pltpu.sync_copy(z16c, cntacc.at[pl.ds(s * stripe, stripe)])
    plsc.subcore_barrier()
    for r in range(_L1_ROWS):
        pltpu.async_copy(h.at[srcbuf.at[r]], rowbuf, sem).wait()
        pltpu.sync_copy(rowbuf, acc.at[dstbuf.at[r]], add=True)
        pltpu.sync_copy(onesbuf, cntacc.at[dstbuf.at[r]], add=True)
    plsc.subcore_barrier()
    pltpu.sync_copy(acc.at[pl.ds(s * stripe, stripe)],
                    s1_out.at[c, pl.ds(s * stripe, stripe)])
    pltpu.sync_copy(cntacc.at[pl.ds(s * stripe, stripe)],
                    cnt_out.at[c, pl.ds(s * stripe, stripe)])


def _l1_agg(h, src, dst, ones, z128, z16c):
    f = functools.partial(
        pl.kernel,
        out_type=(jax.ShapeDtypeStruct((NC, N2, D), jnp.float32),
                  jax.ShapeDtypeStruct((NC, N2, 16), jnp.float32)),
        mesh=_mesh(),
        scratch_types=[
            pltpu.VMEM((_L1_ROWS, G), jnp.int32),
            pltpu.VMEM((_L1_ROWS, G), jnp.int32),
            pltpu.VMEM((G, D), jnp.float32),
            pltpu.VMEM((G, 16), jnp.float32),
            pltpu.VMEM_SHARED((N2, D), jnp.float32),
            pltpu.VMEM_SHARED((N2, 16), jnp.float32),
            pltpu.SemaphoreType.DMA,
        ],
        compiler_params=_SC_PARAMS,
    )(_l1_agg_body)
    return f(h, src, dst, ones, z128, z16c)


def _l1_dense_body(s1_ref, cnt_ref, hd_ref, wl_ref, wr_ref, b_ref, out_ref):
    summed = s1_ref[0] + s1_ref[1]
    cnt = cnt_ref[0][:, 0:1] + cnt_ref[1][:, 0:1]
    mean = summed / jnp.maximum(cnt, 1.0)
    o = jnp.dot(mean, wl_ref[...], preferred_element_type=jnp.float32)
    o += jnp.dot(hd_ref[...], wr_ref[...], preferred_element_type=jnp.float32)
    o += b_ref[...]
    m = jnp.max(o, axis=-1, keepdims=True)
    e = jnp.exp(o - m)
    lse = jnp.log(jnp.sum(e, axis=-1, keepdims=True))
    out_ref[...] = o - m - lse


def _l1_dense(s1, cnt1, h_dst, W_l, W_r, b):
    return pl.pallas_call(
        _l1_dense_body,
        out_shape=jax.ShapeDtypeStruct((N2, D), jnp.float32),
    )(s1, cnt1, h_dst, W_l, W_r, b)


def kernel(x, edge_index_0, edge_index_1, W_l0, b_l0, W_r0, W_l1, b_l1, W_r1):
    x2 = x.reshape(2 * N0, H)
    src0 = edge_index_0[0].astype(jnp.int32).reshape(E0 // G, G)
    dst0 = edge_index_0[1].astype(jnp.int32).reshape(E0 // G, G)
    src1 = edge_index_1[0].astype(jnp.int32).reshape(E1 // G, G)
    dst1 = edge_index_1[1].astype(jnp.int32).reshape(E1 // G, G)
    ones = jnp.ones((G, 16), jnp.float32)
    z64 = jnp.zeros((N1 // NS, H), jnp.float32)
    z16 = jnp.zeros((N1 // NS, 16), jnp.float32)
    z128 = jnp.zeros((N2 // NS, D), jnp.float32)
    z16c = jnp.zeros((N2 // NS, 16), jnp.float32)

    s2, cnt0 = _l0_agg(x2, src0, dst0, ones, z64, z16)
    h = _l0_dense(s2, cnt0, x[:N1], W_l0, W_r0, b_l0.reshape(1, D))
    s1, cnt1 = _l1_agg(h, src1, dst1, ones, z128, z16c)
    return _l1_dense(s1, cnt1, h[:N2], W_l1, W_r1, b_l1.reshape(1, D))


# speedup vs baseline: 7.1546x; 1.0017x over previous
import functools

import jax
import jax.numpy as jnp
from jax import lax
from jax.experimental import pallas as pl
from jax.experimental.pallas import tpu as pltpu
from jax.experimental.pallas import tpu_sc as plsc

N0 = 262144
N1 = 16384
N2 = 1024
E0 = 262144
E1 = 16384
D = 128
H = 64
NC = 2
NS = 16
G = 128

_mesh = lambda: plsc.VectorSubcoreMesh(
    core_axis_name="c", subcore_axis_name="s", num_cores=NC, num_subcores=NS)
_SC_PARAMS = pltpu.CompilerParams(use_tc_tiling_on_sc=False)


_L0_ROWS = E0 // G // NS
_L0_CH = 32
_NBUF = 4


def _l0_agg_body(x2, src, dst, ones, z64, z16, s2_out, cnt_out,
                 srcbuf, dstbuf, rowbuf, onesbuf, acc, cntacc,
                 gsem0, gsem1, gsem2, gsem3, ssem0, ssem1, ssem2, ssem3):
    c = lax.axis_index("c")
    s = lax.axis_index("s")
    base = s * _L0_ROWS
    gsems = (gsem0, gsem1, gsem2, gsem3)
    ssems = (ssem0, ssem1, ssem2, ssem3)
    pltpu.sync_copy(ones, onesbuf)
    stripe = N1 // NS
    pltpu.sync_copy(z64, acc.at[pl.ds(s * stripe, stripe)])
    pltpu.sync_copy(z16, cntacc.at[pl.ds(s * stripe, stripe)])
    plsc.subcore_barrier()

    def chunk(k, carry):
        pltpu.sync_copy(src.at[pl.ds(base + k * _L0_CH, _L0_CH)], srcbuf)
        pltpu.sync_copy(dst.at[pl.ds(base + k * _L0_CH, _L0_CH)], dstbuf)

        def idx_body(r, cc):
            for j in range(G // 16):
                sl = pl.ds(j * 16, 16)
                srcbuf[r, sl] = srcbuf[r, sl] * 2 + c
            return cc

        lax.fori_loop(0, _L0_CH, idx_body, 0)

        for b in range(_NBUF):
            pltpu.async_copy(x2.at[srcbuf.at[b]], rowbuf.at[b], gsems[b])

        def body(g, cc):
            for b in range(_NBUF):
                r = g * _NBUF + b
                pltpu.make_async_copy(
                    x2.at[pl.ds(0, G)], rowbuf.at[b], gsems[b]).wait()
                pltpu.sync_copy(rowbuf.at[b], acc.at[dstbuf.at[r]], add=True)

                @pl.when(c == b % 2)
                def _():
                    pltpu.sync_copy(onesbuf, cntacc.at[dstbuf.at[r]],
                                    add=True)

                @pl.when(r + _NBUF < _L0_CH)
                def _():
                    pltpu.async_copy(
                        x2.at[srcbuf.at[r + _NBUF]], rowbuf.at[b], gsems[b])

            return cc

        lax.fori_loop(0, _L0_CH // _NBUF, body, 0)
        return carry

    lax.fori_loop(0, _L0_ROWS // _L0_CH, chunk, 0)
    plsc.subcore_barrier()
    pltpu.sync_copy(acc.at[pl.ds(s * stripe, stripe)],
                    s2_out.at[c, pl.ds(s * stripe, stripe)])
    pltpu.sync_copy(cntacc.at[pl.ds(s * stripe, stripe)],
                    cnt_out.at[c, pl.ds(s * stripe, stripe)])


def _l0_agg(x2, src, dst, ones, z64, z16):
    f = functools.partial(
        pl.kernel,
        out_type=(jax.ShapeDtypeStruct((NC, N1, H), jnp.float32),
                  jax.ShapeDtypeStruct((NC, N1, 16), jnp.float32)),
        mesh=_mesh(),
        scratch_types=[
            pltpu.VMEM((_L0_CH, G), jnp.int32),
            pltpu.VMEM((_L0_CH, G), jnp.int32),
            pltpu.VMEM((_NBUF, G, H), jnp.float32),
            pltpu.VMEM((G, 16), jnp.float32),
            pltpu.VMEM_SHARED((N1, H), jnp.float32),
            pltpu.VMEM_SHARED((N1, 16), jnp.float32),
            pltpu.SemaphoreType.DMA,
            pltpu.SemaphoreType.DMA,
            pltpu.SemaphoreType.DMA,
            pltpu.SemaphoreType.DMA,
            pltpu.SemaphoreType.DMA,
            pltpu.SemaphoreType.DMA,
            pltpu.SemaphoreType.DMA,
            pltpu.SemaphoreType.DMA,
        ],
        compiler_params=_SC_PARAMS,
    )(_l0_agg_body)
    return f(x2, src, dst, ones, z64, z16)


_B0 = 2048


def _l0_dense_body(s2_ref, cnt_ref, xd_ref, wl_ref, wr_ref, b_ref, out_ref):
    cnt = cnt_ref[0][:, 0:1] + cnt_ref[1][:, 0:1]
    inv = 1.0 / jnp.maximum(cnt, 1.0)
    wl = wl_ref[...]
    acc = jnp.dot(s2_ref[0] * inv, wl[:H], preferred_element_type=jnp.float32)
    acc += jnp.dot(s2_ref[1] * inv, wl[H:], preferred_element_type=jnp.float32)
    acc += jnp.dot(xd_ref[...], wr_ref[...], preferred_element_type=jnp.float32)
    out_ref[...] = jnp.maximum(acc + b_ref[...], 0.0)


def _l0_dense(s2, cnt, x_dst, W_l, W_r, b):
    grid = N1 // _B0
    return pl.pallas_call(
        _l0_dense_body,
        grid=(grid,),
        in_specs=[
            pl.BlockSpec((NC, _B0, H), lambda i: (0, i, 0)),
            pl.BlockSpec((NC, _B0, 16), lambda i: (0, i, 0)),
            pl.BlockSpec((_B0, D), lambda i: (i, 0)),
            pl.BlockSpec((D, D), lambda i: (0, 0)),
            pl.BlockSpec((D, D), lambda i: (0, 0)),
            pl.BlockSpec((1, D), lambda i: (0, 0)),
        ],
        out_specs=pl.BlockSpec((_B0, D), lambda i: (i, 0)),
        out_shape=jax.ShapeDtypeStruct((N1, D), jnp.float32),
    )(s2, cnt, x_dst, W_l, W_r, b)


_L1_ROWS = E1 // G // (NC * NS)


def _l1_agg_body(h, src, dst, ones, z128, z16c, s1_out, cnt_out,
                 srcbuf, dstbuf, rowbuf, onesbuf, acc, cntacc, sem):
    c = lax.axis_index("c")
    s = lax.axis_index("s")
    w = s * NC + c
    pltpu.sync_copy(src.at[pl.ds(w * _L1_ROWS, _L1_ROWS)], srcbuf)
    pltpu.sync_copy(dst.at[pl.ds(w * _L1_ROWS, _L1_ROWS)], dstbuf)
    pltpu.sync_copy(ones, onesbuf)
    stripe = N2 // NS
    pltpu.sync_copy(z128, acc.at[pl.ds(s * stripe, stripe)])
    pltpu.sync_copy(z16c, cntacc.at[pl.ds(s * stripe, stripe)])
    plsc.subcore_barrier()
    for r in range(_L1_ROWS):
        pltpu.async_copy(h.at[srcbuf.at[r]], rowbuf, sem).wait()
        pltpu.sync_copy(rowbuf, acc.at[dstbuf.at[r]], add=True)
        pltpu.sync_copy(onesbuf, cntacc.at[dstbuf.at[r]], add=True)
    plsc.subcore_barrier()
    pltpu.sync_copy(acc.at[pl.ds(s * stripe, stripe)],
                    s1_out.at[c, pl.ds(s * stripe, stripe)])
    pltpu.sync_copy(cntacc.at[pl.ds(s * stripe, stripe)],
                    cnt_out.at[c, pl.ds(s * stripe, stripe)])


def _l1_agg(h, src, dst, ones, z128, z16c):
    f = functools.partial(
        pl.kernel,
        out_type=(jax.ShapeDtypeStruct((NC, N2, D), jnp.float32),
                  jax.ShapeDtypeStruct((NC, N2, 16), jnp.float32)),
        mesh=_mesh(),
        scratch_types=[
            pltpu.VMEM((_L1_ROWS, G), jnp.int32),
            pltpu.VMEM((_L1_ROWS, G), jnp.int32),
            pltpu.VMEM((G, D), jnp.float32),
            pltpu.VMEM((G, 16), jnp.float32),
            pltpu.VMEM_SHARED((N2, D), jnp.float32),
            pltpu.VMEM_SHARED((N2, 16), jnp.float32),
            pltpu.SemaphoreType.DMA,
        ],
        compiler_params=_SC_PARAMS,
    )(_l1_agg_body)
    return f(h, src, dst, ones, z128, z16c)


def _l1_dense_body(s1_ref, cnt_ref, hd_ref, wl_ref, wr_ref, b_ref, out_ref):
    summed = s1_ref[0] + s1_ref[1]
    cnt = cnt_ref[0][:, 0:1] + cnt_ref[1][:, 0:1]
    mean = summed / jnp.maximum(cnt, 1.0)
    o = jnp.dot(mean, wl_ref[...], preferred_element_type=jnp.float32)
    o += jnp.dot(hd_ref[...], wr_ref[...], preferred_element_type=jnp.float32)
    o += b_ref[...]
    m = jnp.max(o, axis=-1, keepdims=True)
    e = jnp.exp(o - m)
    lse = jnp.log(jnp.sum(e, axis=-1, keepdims=True))
    out_ref[...] = o - m - lse


def _l1_dense(s1, cnt1, h_dst, W_l, W_r, b):
    return pl.pallas_call(
        _l1_dense_body,
        out_shape=jax.ShapeDtypeStruct((N2, D), jnp.float32),
    )(s1, cnt1, h_dst, W_l, W_r, b)


def kernel(x, edge_index_0, edge_index_1, W_l0, b_l0, W_r0, W_l1, b_l1, W_r1):
    x2 = x.reshape(2 * N0, H)
    src0 = edge_index_0[0].astype(jnp.int32).reshape(E0 // G, G)
    dst0 = edge_index_0[1].astype(jnp.int32).reshape(E0 // G, G)
    src1 = edge_index_1[0].astype(jnp.int32).reshape(E1 // G, G)
    dst1 = edge_index_1[1].astype(jnp.int32).reshape(E1 // G, G)
    ones = jnp.ones((G, 16), jnp.float32)
    z64 = jnp.zeros((N1 // NS, H), jnp.float32)
    z16 = jnp.zeros((N1 // NS, 16), jnp.float32)
    z128 = jnp.zeros((N2 // NS, D), jnp.float32)
    z16c = jnp.zeros((N2 // NS, 16), jnp.float32)

    s2, cnt0 = _l0_agg(x2, src0, dst0, ones, z64, z16)
    h = _l0_dense(s2, cnt0, x[:N1], W_l0, W_r0, b_l0.reshape(1, D))
    s1, cnt1 = _l1_agg(h, src1, dst1, ones, z128, z16c)
    return _l1_dense(s1, cnt1, h[:N2], W_l1, W_r1, b_l1.reshape(1, D))
